# baseline (speedup 1.0000x reference)
import functools

import jax
import jax.numpy as jnp
from jax import lax
from jax.experimental import pallas as pl
from jax.experimental.pallas import tpu as pltpu
from jax.experimental.pallas import tpu_sc as plsc

_B, _T, _D = 4096, 125, 128
_TP = 128
_V = 1000
_NC, _NS = 2, 16
_NW = _NC * _NS
_NCHUNK = 4
_BC = _B // _NCHUNK
_BPW = _BC // _NW
_LANES = 16
_DV = _D // _LANES
_NBUF = 4
_BT = 16

_mesh = plsc.VectorSubcoreMesh(
    core_axis_name="c", subcore_axis_name="s", num_cores=_NC, num_subcores=_NS
)

_SEQ_CHUNK = jax.ShapeDtypeStruct((_BC, _TP, _D), jnp.float32)
_ZS_OUT = jax.ShapeDtypeStruct((_B, 1, _D), jnp.float32)
_SCRATCH = [
    pltpu.VMEM_SHARED((_V, _D), jnp.float32),
    pltpu.VMEM((_BPW, _T), jnp.int32),
    pltpu.VMEM((_T, _D), jnp.float32),
    pltpu.VMEM((3, _D), jnp.float32),
    [pltpu.VMEM((_TP, _D), jnp.float32)] * _NBUF,
    [pltpu.SemaphoreType.DMA] * _NBUF,
    [pltpu.SemaphoreType.DMA] * _NBUF,
]


def _chunk_work(chunk, idx_hbm, table, te_hbm, pe_hbm, out_seq,
                table_sh, idx_v, bias_v, te_v, rows, gsem, osem):
    sid = lax.axis_index("s")
    wid = sid * _NC + lax.axis_index("c")
    gbase = chunk * _BC + wid * _BPW
    obase = wid * _BPW

    @pl.when(sid == 0)
    def _():
        pltpu.sync_copy(table, table_sh)

    pltpu.sync_copy(te_hbm, te_v)
    pltpu.sync_copy(pe_hbm, bias_v)
    pltpu.sync_copy(idx_hbm.at[pl.ds(gbase, _BPW)], idx_v)

    def bias_body(t5, carry):
        for u in range(5):
            t = t5 * 5 + u
            for d in range(_DV):
                sl = pl.ds(d * _LANES, _LANES)
                plsc.addupdate(bias_v.at[t, sl], te_v[2, sl])
        return carry
    lax.fori_loop(0, _T // 5, bias_body, 0)

    plsc.subcore_barrier()

    def g_copy(k, j):
        return pltpu.make_async_copy(
            table_sh.at[idx_v.at[k]], rows[j].at[pl.ds(0, _T)], gsem[j])

    def o_copy(k, j):
        return pltpu.make_async_copy(rows[j], out_seq.at[obase + k], osem[j])

    def add_bias(k, j):
        def add_body(t5, carry):
            for u in range(5):
                t = t5 * 5 + u
                for d in range(_DV):
                    sl = pl.ds(d * _LANES, _LANES)
                    plsc.addupdate(rows[j].at[t, sl], bias_v[t, sl])
            return carry
        lax.fori_loop(0, _T // 5, add_body, 0)

    g_copy(0, 0).start()
    g_copy(1, 1).start()
    for k in (0, 1):
        g_copy(k, k).wait()
        add_bias(k, k)
        o_copy(k, k).start()
        g_copy(k + 2, k + 2).start()

    def main_body(k4, carry):
        for j in range(_NBUF):
            k = 2 + k4 * _NBUF + j
            buf = (2 + j) % _NBUF
            nbuf = j % _NBUF
            g_copy(k, buf).wait()
            add_bias(k, buf)
            o_copy(k, buf).start()
            o_copy(k - 2, nbuf).wait()
            g_copy(k + 2, nbuf).start()
        return carry
    lax.fori_loop(0, (_BPW - _NBUF) // _NBUF, main_body, 0)

    for k in (_BPW - 2, _BPW - 1):
        j = k % _NBUF
        g_copy(k, j).wait()
        add_bias(k, j)
        o_copy(k, j).start()
    for k in range(_BPW - _NBUF, _BPW):
        o_copy(k, k % _NBUF).wait()


@functools.partial(
    pl.kernel,
    out_type=(_SEQ_CHUNK, _ZS_OUT, _ZS_OUT),
    mesh=_mesh,
    scratch_types=_SCRATCH + [pltpu.VMEM((_B // _NW, 1, _D), jnp.float32)],
    name="embed_chunk0",
)
def _embed_sc_first(zeo, syn, idx_hbm, table, te_hbm, pe_hbm,
                    out_seq, out_zeo, out_syn,
                    table_sh, idx_v, bias_v, te_v, rows, gsem, osem, zs_v):
    wid = lax.axis_index("s") * _NC + lax.axis_index("c")
    zrows = _B // _NW
    zbase = wid * zrows
    pltpu.sync_copy(te_hbm, te_v)
    for src, dst, row in ((zeo, out_zeo, 0), (syn, out_syn, 1)):
        pltpu.sync_copy(src.at[pl.ds(zbase, zrows)], zs_v)
        def zs_body(i, carry, row=row):
            for d in range(_DV):
                sl = pl.ds(d * _LANES, _LANES)
                plsc.addupdate(zs_v.at[i, 0, sl], te_v[row, sl])
            return carry
        lax.fori_loop(0, zrows, zs_body, 0)
        pltpu.sync_copy(zs_v, dst.at[pl.ds(zbase, zrows)])

    _chunk_work(0, idx_hbm, table, te_hbm, pe_hbm, out_seq,
                table_sh, idx_v, bias_v, te_v, rows, gsem, osem)


def _make_rest(chunk):
    @functools.partial(
        pl.kernel,
        out_type=_SEQ_CHUNK,
        mesh=_mesh,
        scratch_types=_SCRATCH,
        name=f"embed_chunk{chunk}",
    )
    def _embed_sc_rest(idx_hbm, table, te_hbm, pe_hbm, out_seq,
                       table_sh, idx_v, bias_v, te_v, rows, gsem, osem):
        _chunk_work(chunk, idx_hbm, table, te_hbm, pe_hbm, out_seq,
                    table_sh, idx_v, bias_v, te_v, rows, gsem, osem)
    return _embed_sc_rest


_REST = [_make_rest(c) for c in range(1, _NCHUNK)]


def _copy_body0(chunk_ref, out_ref):
    out_ref[...] = chunk_ref[...]


def _copy_body(acc_ref, chunk_ref, out_ref):
    del acc_ref
    out_ref[...] = chunk_ref[...]


def _make_tc_copy(chunk):
    blocks = _BC // _BT
    tblocks = _TP // 8
    chunk_spec = pl.BlockSpec((_BT, 8, _D), lambda i, j: (i, j, 0))
    out_spec = pl.BlockSpec(
        (_BT, 8, _D), lambda i, j, c=chunk: (c * (_BC // _BT) + i, j, 0))
    if chunk == 0:
        return pl.pallas_call(
            _copy_body0,
            out_shape=jax.ShapeDtypeStruct((_B, _T, _D), jnp.float32),
            grid=(blocks, tblocks),
            in_specs=[chunk_spec],
            out_specs=out_spec,
            name="relayout_chunk0",
        )
    return pl.pallas_call(
        _copy_body,
        out_shape=jax.ShapeDtypeStruct((_B, _T, _D), jnp.float32),
        grid=(blocks, tblocks),
        in_specs=[
            pl.BlockSpec(memory_space=pltpu.MemorySpace.HBM),
            chunk_spec,
        ],
        out_specs=out_spec,
        input_output_aliases={0: 0},
        name=f"relayout_chunk{chunk}",
    )


_TC_COPY = [_make_tc_copy(c) for c in range(_NCHUNK)]


def kernel(zeo, syn, smis_seq, char_embed, type_embed, pe):
    idx = smis_seq.astype(jnp.int32)
    pe2d = pe.reshape(_T, _D)
    chunk0, out_zeo, out_syn = _embed_sc_first(
        zeo, syn, idx, char_embed, type_embed, pe2d)
    chunks = [chunk0]
    for fn in _REST:
        chunks.append(fn(idx, char_embed, type_embed, pe2d))
    acc = _TC_COPY[0](chunks[0])
    for c in range(1, _NCHUNK):
        acc = _TC_COPY[c](acc, chunks[c])
    return acc, out_zeo, out_syn


# speedup vs baseline: 3.9887x; 3.9887x over previous
import functools

import jax
import jax.numpy as jnp
from jax import lax
from jax.experimental import pallas as pl
from jax.experimental.pallas import tpu as pltpu
from jax.experimental.pallas import tpu_sc as plsc

_B, _T, _D = 4096, 125, 128
_TP = 128
_V = 1000
_NC, _NS = 2, 16
_NW = _NC * _NS
_NCHUNK = 4
_BC = _B // _NCHUNK
_BPW = _BC // _NW
_LANES = 16
_DV = _D // _LANES
_NBUF = 4
_BT = 256

_mesh = plsc.VectorSubcoreMesh(
    core_axis_name="c", subcore_axis_name="s", num_cores=_NC, num_subcores=_NS
)

_SEQ_CHUNK = jax.ShapeDtypeStruct((_BC, _TP, _D), jnp.float32)
_ZS_OUT = jax.ShapeDtypeStruct((_B, 1, _D), jnp.float32)
_SCRATCH = [
    pltpu.VMEM_SHARED((_V, _D), jnp.float32),
    pltpu.VMEM((_BPW, _T), jnp.int32),
    pltpu.VMEM((_T, _D), jnp.float32),
    pltpu.VMEM((3, _D), jnp.float32),
    [pltpu.VMEM((_TP, _D), jnp.float32)] * _NBUF,
    [pltpu.SemaphoreType.DMA] * _NBUF,
    [pltpu.SemaphoreType.DMA] * _NBUF,
]


def _chunk_work(chunk, idx_hbm, table, te_hbm, pe_hbm, out_seq,
                table_sh, idx_v, bias_v, te_v, rows, gsem, osem):
    sid = lax.axis_index("s")
    wid = sid * _NC + lax.axis_index("c")
    gbase = chunk * _BC + wid * _BPW
    obase = wid * _BPW

    @pl.when(sid == 0)
    def _():
        pltpu.sync_copy(table, table_sh)

    pltpu.sync_copy(te_hbm, te_v)
    pltpu.sync_copy(pe_hbm, bias_v)
    pltpu.sync_copy(idx_hbm.at[pl.ds(gbase, _BPW)], idx_v)

    def bias_body(t5, carry):
        for u in range(5):
            t = t5 * 5 + u
            for d in range(_DV):
                sl = pl.ds(d * _LANES, _LANES)
                plsc.addupdate(bias_v.at[t, sl], te_v[2, sl])
        return carry
    lax.fori_loop(0, _T // 5, bias_body, 0)

    plsc.subcore_barrier()

    def g_copy(k, j):
        return pltpu.make_async_copy(
            table_sh.at[idx_v.at[k]], rows[j].at[pl.ds(0, _T)], gsem[j])

    def o_copy(k, j):
        return pltpu.make_async_copy(rows[j], out_seq.at[obase + k], osem[j])

    def add_bias(k, j):
        def add_body(t5, carry):
            for u in range(5):
                t = t5 * 5 + u
                for d in range(_DV):
                    sl = pl.ds(d * _LANES, _LANES)
                    plsc.addupdate(rows[j].at[t, sl], bias_v[t, sl])
            return carry
        lax.fori_loop(0, _T // 5, add_body, 0)

    g_copy(0, 0).start()
    g_copy(1, 1).start()
    for k in (0, 1):
        g_copy(k, k).wait()
        add_bias(k, k)
        o_copy(k, k).start()
        g_copy(k + 2, k + 2).start()

    def main_body(k4, carry):
        for j in range(_NBUF):
            k = 2 + k4 * _NBUF + j
            buf = (2 + j) % _NBUF
            nbuf = j % _NBUF
            g_copy(k, buf).wait()
            add_bias(k, buf)
            o_copy(k, buf).start()
            o_copy(k - 2, nbuf).wait()
            g_copy(k + 2, nbuf).start()
        return carry
    lax.fori_loop(0, (_BPW - _NBUF) // _NBUF, main_body, 0)

    for k in (_BPW - 2, _BPW - 1):
        j = k % _NBUF
        g_copy(k, j).wait()
        add_bias(k, j)
        o_copy(k, j).start()
    for k in range(_BPW - _NBUF, _BPW):
        o_copy(k, k % _NBUF).wait()


@functools.partial(
    pl.kernel,
    out_type=(_SEQ_CHUNK, _ZS_OUT, _ZS_OUT),
    mesh=_mesh,
    scratch_types=_SCRATCH + [pltpu.VMEM((_B // _NW, 1, _D), jnp.float32)],
    name="embed_chunk0",
)
def _embed_sc_first(zeo, syn, idx_hbm, table, te_hbm, pe_hbm,
                    out_seq, out_zeo, out_syn,
                    table_sh, idx_v, bias_v, te_v, rows, gsem, osem, zs_v):
    wid = lax.axis_index("s") * _NC + lax.axis_index("c")
    zrows = _B // _NW
    zbase = wid * zrows
    pltpu.sync_copy(te_hbm, te_v)
    for src, dst, row in ((zeo, out_zeo, 0), (syn, out_syn, 1)):
        pltpu.sync_copy(src.at[pl.ds(zbase, zrows)], zs_v)
        def zs_body(i, carry, row=row):
            for d in range(_DV):
                sl = pl.ds(d * _LANES, _LANES)
                plsc.addupdate(zs_v.at[i, 0, sl], te_v[row, sl])
            return carry
        lax.fori_loop(0, zrows, zs_body, 0)
        pltpu.sync_copy(zs_v, dst.at[pl.ds(zbase, zrows)])

    _chunk_work(0, idx_hbm, table, te_hbm, pe_hbm, out_seq,
                table_sh, idx_v, bias_v, te_v, rows, gsem, osem)


def _make_rest(chunk):
    @functools.partial(
        pl.kernel,
        out_type=_SEQ_CHUNK,
        mesh=_mesh,
        scratch_types=_SCRATCH,
        name=f"embed_chunk{chunk}",
    )
    def _embed_sc_rest(idx_hbm, table, te_hbm, pe_hbm, out_seq,
                       table_sh, idx_v, bias_v, te_v, rows, gsem, osem):
        _chunk_work(chunk, idx_hbm, table, te_hbm, pe_hbm, out_seq,
                    table_sh, idx_v, bias_v, te_v, rows, gsem, osem)
    return _embed_sc_rest


_REST = [_make_rest(c) for c in range(1, _NCHUNK)]


def _copy_body0(chunk_ref, out_ref):
    out_ref[...] = chunk_ref[...]


def _copy_body(acc_ref, chunk_ref, out_ref):
    del acc_ref
    out_ref[...] = chunk_ref[...]


def _make_tc_copy(chunk):
    blocks = _BC // _BT
    tblocks = _TP // 8
    chunk_spec = pl.BlockSpec((_BT, 8, _D), lambda i, j: (i, j, 0))
    out_spec = pl.BlockSpec(
        (_BT, 8, _D), lambda i, j, c=chunk: (c * (_BC // _BT) + i, j, 0))
    if chunk == 0:
        return pl.pallas_call(
            _copy_body0,
            out_shape=jax.ShapeDtypeStruct((_B, _T, _D), jnp.float32),
            grid=(blocks, tblocks),
            in_specs=[chunk_spec],
            out_specs=out_spec,
            name="relayout_chunk0",
        )
    return pl.pallas_call(
        _copy_body,
        out_shape=jax.ShapeDtypeStruct((_B, _T, _D), jnp.float32),
        grid=(blocks, tblocks),
        in_specs=[
            pl.BlockSpec(memory_space=pltpu.MemorySpace.HBM),
            chunk_spec,
        ],
        out_specs=out_spec,
        input_output_aliases={0: 0},
        name=f"relayout_chunk{chunk}",
    )


_TC_COPY = [_make_tc_copy(c) for c in range(_NCHUNK)]


def kernel(zeo, syn, smis_seq, char_embed, type_embed, pe):
    idx = smis_seq.astype(jnp.int32)
    pe2d = pe.reshape(_T, _D)
    chunk0, out_zeo, out_syn = _embed_sc_first(
        zeo, syn, idx, char_embed, type_embed, pe2d)
    chunks = [chunk0]
    for fn in _REST:
        chunks.append(fn(idx, char_embed, type_embed, pe2d))
    acc = _TC_COPY[0](chunks[0])
    for c in range(1, _NCHUNK):
        acc = _TC_COPY[c](acc, chunks[c])
    return acc, out_zeo, out_syn


# speedup vs baseline: 4.2242x; 1.0590x over previous
import functools

import jax
import jax.numpy as jnp
from jax import lax
from jax.experimental import pallas as pl
from jax.experimental.pallas import tpu as pltpu
from jax.experimental.pallas import tpu_sc as plsc

_B, _T, _D = 4096, 125, 128
_TP = 128
_V = 1000
_NC, _NS = 2, 16
_NW = _NC * _NS
_NCHUNK = 4
_BC = _B // _NCHUNK
_BPW = _BC // _NW
_LANES = 16
_DV = _D // _LANES
_NBUF = 4
_BT = 256

_mesh = plsc.VectorSubcoreMesh(
    core_axis_name="c", subcore_axis_name="s", num_cores=_NC, num_subcores=_NS
)

_SEQ_CHUNK = jax.ShapeDtypeStruct((_BC, _TP, _D), jnp.float32)
_ZS_OUT = jax.ShapeDtypeStruct((_B, 1, _D), jnp.float32)
_SCRATCH = [
    pltpu.VMEM_SHARED((_V, _D), jnp.float32),
    pltpu.VMEM((_BPW, _T), jnp.int32),
    pltpu.VMEM((_T, _D), jnp.float32),
    pltpu.VMEM((3, _D), jnp.float32),
    [pltpu.VMEM((_TP, _D), jnp.float32)] * _NBUF,
    [pltpu.SemaphoreType.DMA] * _NBUF,
    [pltpu.SemaphoreType.DMA] * _NBUF,
]


def _chunk_work(chunk, idx_hbm, table, te_hbm, pe_hbm, out_seq,
                table_sh, idx_v, bias_v, te_v, rows, gsem, osem):
    sid = lax.axis_index("s")
    wid = sid * _NC + lax.axis_index("c")
    gbase = chunk * _BC + wid * _BPW
    obase = wid * _BPW

    @pl.when(sid == 0)
    def _():
        pltpu.sync_copy(table, table_sh)

    pltpu.sync_copy(te_hbm, te_v)
    pltpu.sync_copy(pe_hbm, bias_v)
    pltpu.sync_copy(idx_hbm.at[pl.ds(gbase, _BPW)], idx_v)

    def bias_body(t5, carry):
        for u in range(5):
            t = t5 * 5 + u
            for d in range(_DV):
                sl = pl.ds(d * _LANES, _LANES)
                plsc.addupdate(bias_v.at[t, sl], te_v[2, sl])
        return carry
    lax.fori_loop(0, _T // 5, bias_body, 0)

    plsc.subcore_barrier()

    def g_copy(k, j):
        return pltpu.make_async_copy(
            table_sh.at[idx_v.at[k]], rows[j].at[pl.ds(0, _T)], gsem[j])

    def o_copy(k, j):
        return pltpu.make_async_copy(rows[j], out_seq.at[obase + k], osem[j])

    def add_bias(k, j):
        def add_body(t5, carry):
            for u in range(5):
                t = t5 * 5 + u
                for d in range(_DV):
                    sl = pl.ds(d * _LANES, _LANES)
                    plsc.addupdate(rows[j].at[t, sl], bias_v[t, sl])
            return carry
        lax.fori_loop(0, _T // 5, add_body, 0)

    g_copy(0, 0).start()
    g_copy(1, 1).start()
    for k in (0, 1):
        g_copy(k, k).wait()
        add_bias(k, k)
        o_copy(k, k).start()
        g_copy(k + 2, k + 2).start()

    def main_body(k4, carry):
        for j in range(_NBUF):
            k = 2 + k4 * _NBUF + j
            buf = (2 + j) % _NBUF
            nbuf = j % _NBUF
            g_copy(k, buf).wait()
            add_bias(k, buf)
            o_copy(k, buf).start()
            o_copy(k - 2, nbuf).wait()
            g_copy(k + 2, nbuf).start()
        return carry
    lax.fori_loop(0, (_BPW - _NBUF) // _NBUF, main_body, 0)

    for k in (_BPW - 2, _BPW - 1):
        j = k % _NBUF
        g_copy(k, j).wait()
        add_bias(k, j)
        o_copy(k, j).start()
    for k in range(_BPW - _NBUF, _BPW):
        o_copy(k, k % _NBUF).wait()


@functools.partial(
    pl.kernel,
    out_type=(_SEQ_CHUNK, _ZS_OUT, _ZS_OUT),
    mesh=_mesh,
    scratch_types=_SCRATCH + [pltpu.VMEM((_B // _NW, 1, _D), jnp.float32)],
    name="embed_chunk0",
)
def _embed_sc_first(zeo, syn, idx_hbm, table, te_hbm, pe_hbm,
                    out_seq, out_zeo, out_syn,
                    table_sh, idx_v, bias_v, te_v, rows, gsem, osem, zs_v):
    wid = lax.axis_index("s") * _NC + lax.axis_index("c")
    zrows = _B // _NW
    zbase = wid * zrows
    pltpu.sync_copy(te_hbm, te_v)
    for src, dst, row in ((zeo, out_zeo, 0), (syn, out_syn, 1)):
        pltpu.sync_copy(src.at[pl.ds(zbase, zrows)], zs_v)
        def zs_body(i, carry, row=row):
            for d in range(_DV):
                sl = pl.ds(d * _LANES, _LANES)
                plsc.addupdate(zs_v.at[i, 0, sl], te_v[row, sl])
            return carry
        lax.fori_loop(0, zrows, zs_body, 0)
        pltpu.sync_copy(zs_v, dst.at[pl.ds(zbase, zrows)])

    _chunk_work(0, idx_hbm, table, te_hbm, pe_hbm, out_seq,
                table_sh, idx_v, bias_v, te_v, rows, gsem, osem)


def _make_rest(chunk):
    @functools.partial(
        pl.kernel,
        out_type=_SEQ_CHUNK,
        mesh=_mesh,
        scratch_types=_SCRATCH,
        name=f"embed_chunk{chunk}",
    )
    def _embed_sc_rest(idx_hbm, table, te_hbm, pe_hbm, out_seq,
                       table_sh, idx_v, bias_v, te_v, rows, gsem, osem):
        _chunk_work(chunk, idx_hbm, table, te_hbm, pe_hbm, out_seq,
                    table_sh, idx_v, bias_v, te_v, rows, gsem, osem)
    return _embed_sc_rest


_REST = [_make_rest(c) for c in range(1, _NCHUNK)]


def kernel(zeo, syn, smis_seq, char_embed, type_embed, pe):
    idx = smis_seq.astype(jnp.int32)
    pe2d = pe.reshape(_T, _D)
    chunk0, out_zeo, out_syn = _embed_sc_first(
        zeo, syn, idx, char_embed, type_embed, pe2d)
    chunks = [chunk0]
    for fn in _REST:
        chunks.append(fn(idx, char_embed, type_embed, pe2d))
    acc = jnp.zeros((_B, _T, _D), jnp.float32)
    for c in range(_NCHUNK):
        acc = lax.dynamic_update_slice(
            acc, chunks[c][:, :_T, :], (c * _BC, 0, 0))
        acc = lax.optimization_barrier(acc)
    return acc, out_zeo, out_syn


# speedup vs baseline: 5.8323x; 1.3807x over previous
import functools

import jax
import jax.numpy as jnp
from jax import lax
from jax.experimental import pallas as pl
from jax.experimental.pallas import tpu as pltpu
from jax.experimental.pallas import tpu_sc as plsc

_B, _T, _D = 4096, 125, 128
_TP = 128
_V = 1000
_NC, _NS = 2, 16
_NW = _NC * _NS
_BPW = _B // _NW
_LANES = 16
_DV = _D // _LANES
_NBUF = 4

_mesh = plsc.VectorSubcoreMesh(
    core_axis_name="c", subcore_axis_name="s", num_cores=_NC, num_subcores=_NS
)


@functools.partial(
    pl.kernel,
    out_type=(
        jax.ShapeDtypeStruct((_B, _T, _D), jnp.float32),
        jax.ShapeDtypeStruct((_B, _D), jnp.float32),
        jax.ShapeDtypeStruct((_B, _D), jnp.float32),
    ),
    mesh=_mesh,
    compiler_params=pltpu.CompilerParams(use_tc_tiling_on_sc=True),
    scratch_types=[
        pltpu.VMEM_SHARED((_V, _D), jnp.float32),
        pltpu.VMEM((_BPW, _TP), jnp.int32),
        pltpu.VMEM((_TP, _D), jnp.float32),
        pltpu.VMEM((8, _D), jnp.float32),
        [pltpu.VMEM((_TP, _D), jnp.float32)] * _NBUF,
        pltpu.VMEM((_BPW, _D), jnp.float32),
        [pltpu.SemaphoreType.DMA] * _NBUF,
        [pltpu.SemaphoreType.DMA] * _NBUF,
    ],
)
def _embed_sc(zeo, syn, idx_hbm, table, te_hbm, pe_hbm,
              out_seq, out_zeo, out_syn,
              table_sh, idx_v, bias_v, te_v, rows, zs_v, gsem, osem):
    sid = lax.axis_index("s")
    wid = sid * _NC + lax.axis_index("c")
    base = wid * _BPW

    @pl.when(sid == 0)
    def _():
        pltpu.sync_copy(table, table_sh)

    pltpu.sync_copy(te_hbm, te_v)
    pltpu.sync_copy(pe_hbm, bias_v)
    pltpu.sync_copy(idx_hbm.at[pl.ds(base, _BPW)], idx_v)

    def bias_body(t5, carry):
        for u in range(5):
            t = t5 * 5 + u
            for d in range(_DV):
                sl = pl.ds(d * _LANES, _LANES)
                plsc.addupdate(bias_v.at[t, sl], te_v[2, sl])
        return carry
    lax.fori_loop(0, _T // 5, bias_body, 0)

    for src, dst, row in ((zeo, out_zeo, 0), (syn, out_syn, 1)):
        pltpu.sync_copy(src.at[pl.ds(base, _BPW)], zs_v)
        def zs_body(i, carry, row=row):
            for d in range(_DV):
                sl = pl.ds(d * _LANES, _LANES)
                plsc.addupdate(zs_v.at[i, sl], te_v[row, sl])
            return carry
        lax.fori_loop(0, _BPW, zs_body, 0)
        pltpu.sync_copy(zs_v, dst.at[pl.ds(base, _BPW)])

    plsc.subcore_barrier()

    def g_copy(k, j):
        return pltpu.make_async_copy(
            table_sh.at[idx_v.at[k]], rows[j], gsem[j])

    def o_copy(k, j):
        return pltpu.make_async_copy(
            rows[j].at[pl.ds(0, _T)], out_seq.at[base + k], osem[j])

    def add_bias(k, j):
        def add_body(t5, carry):
            for u in range(5):
                t = t5 * 5 + u
                for d in range(_DV):
                    sl = pl.ds(d * _LANES, _LANES)
                    plsc.addupdate(rows[j].at[t, sl], bias_v[t, sl])
            return carry
        lax.fori_loop(0, _T // 5, add_body, 0)

    g_copy(0, 0).start()
    g_copy(1, 1).start()
    for k in (0, 1):
        g_copy(k, k).wait()
        add_bias(k, k)
        o_copy(k, k).start()
        g_copy(k + 2, k + 2).start()

    def main_body(k4, carry):
        for j in range(_NBUF):
            k = 2 + k4 * _NBUF + j
            buf = (2 + j) % _NBUF
            nbuf = j % _NBUF
            g_copy(k, buf).wait()
            add_bias(k, buf)
            o_copy(k, buf).start()
            o_copy(k - 2, nbuf).wait()
            g_copy(k + 2, nbuf).start()
        return carry
    lax.fori_loop(0, (_BPW - _NBUF) // _NBUF, main_body, 0)

    for k in (_BPW - 2, _BPW - 1):
        j = k % _NBUF
        g_copy(k, j).wait()
        add_bias(k, j)
        o_copy(k, j).start()
    for k in range(_BPW - _NBUF, _BPW):
        o_copy(k, k % _NBUF).wait()


def kernel(zeo, syn, smis_seq, char_embed, type_embed, pe):
    idx = jnp.pad(smis_seq.astype(jnp.int32), ((0, 0), (0, _TP - _T)))
    pe_pad = jnp.pad(pe.reshape(_T, _D), ((0, _TP - _T), (0, 0)))
    te_pad = jnp.pad(type_embed, ((0, 8 - type_embed.shape[0]), (0, 0)))
    zeo2d = zeo.reshape(_B, _D)
    syn2d = syn.reshape(_B, _D)
    out_seq, out_zeo, out_syn = _embed_sc(
        zeo2d, syn2d, idx, char_embed, te_pad, pe_pad)
    return out_seq, out_zeo.reshape(_B, 1, _D), out_syn.reshape(_B, 1, _D)


# speedup vs baseline: 13.6633x; 2.3427x over previous
import functools

import jax
import jax.numpy as jnp
from jax import lax
from jax.experimental import pallas as pl
from jax.experimental.pallas import tpu as pltpu
from jax.experimental.pallas import tpu_sc as plsc

_B, _T, _D = 4096, 125, 128
_V = 1000
_NC, _NS = 2, 16
_NW = _NC * _NS
_BPW = _B // _NW
_LANES = 16
_DV = _D // _LANES
_NBUF = 4

_mesh = plsc.VectorSubcoreMesh(
    core_axis_name="c", subcore_axis_name="s", num_cores=_NC, num_subcores=_NS
)


@functools.partial(
    pl.kernel,
    out_type=(
        jax.ShapeDtypeStruct((_T, _B, _D), jnp.float32),
        jax.ShapeDtypeStruct((_B, 1, _D), jnp.float32),
        jax.ShapeDtypeStruct((_B, 1, _D), jnp.float32),
    ),
    mesh=_mesh,
    scratch_types=[
        pltpu.VMEM_SHARED((_V, _D), jnp.float32),
        pltpu.VMEM((_T, _BPW), jnp.int32),
        pltpu.VMEM((_T, _D), jnp.float32),
        pltpu.VMEM((3, _D), jnp.float32),
        [pltpu.VMEM((_BPW, _D), jnp.float32)] * _NBUF,
        pltpu.VMEM((_BPW, 1, _D), jnp.float32),
        [pltpu.SemaphoreType.DMA] * _NBUF,
        [pltpu.SemaphoreType.DMA] * _NBUF,
    ],
)
def _embed_sc(zeo, syn, idxt_hbm, table, te_hbm, pe_hbm,
              out_seq, out_zeo, out_syn,
              table_sh, idx_v, bias_v, te_v, rows, zs_v, gsem, osem):
    sid = lax.axis_index("s")
    wid = sid * _NC + lax.axis_index("c")
    base = wid * _BPW

    @pl.when(sid == 0)
    def _():
        pltpu.sync_copy(table, table_sh)

    pltpu.sync_copy(te_hbm, te_v)
    pltpu.sync_copy(pe_hbm, bias_v)
    pltpu.sync_copy(idxt_hbm.at[:, pl.ds(base, _BPW)], idx_v)

    def bias_body(t5, carry):
        for u in range(5):
            t = t5 * 5 + u
            for d in range(_DV):
                sl = pl.ds(d * _LANES, _LANES)
                plsc.addupdate(bias_v.at[t, sl], te_v[2, sl])
        return carry
    lax.fori_loop(0, _T // 5, bias_body, 0)

    for src, dst, row in ((zeo, out_zeo, 0), (syn, out_syn, 1)):
        pltpu.sync_copy(src.at[pl.ds(base, _BPW)], zs_v)
        def zs_body(i, carry, row=row):
            for d in range(_DV):
                sl = pl.ds(d * _LANES, _LANES)
                plsc.addupdate(zs_v.at[i, 0, sl], te_v[row, sl])
            return carry
        lax.fori_loop(0, _BPW, zs_body, 0)
        pltpu.sync_copy(zs_v, dst.at[pl.ds(base, _BPW)])

    plsc.subcore_barrier()

    def g_copy(t, j):
        return pltpu.make_async_copy(
            table_sh.at[idx_v.at[t]], rows[j], gsem[j])

    def o_copy(t, j):
        return pltpu.make_async_copy(
            rows[j], out_seq.at[t, pl.ds(base, _BPW)], osem[j])

    def add_bias(t, j):
        bias_regs = [bias_v[t, pl.ds(d * _LANES, _LANES)] for d in range(_DV)]
        def add_body(r4, carry):
            for rr in range(4):
                r = r4 * 4 + rr
                for d in range(_DV):
                    sl = pl.ds(d * _LANES, _LANES)
                    plsc.addupdate(rows[j].at[r, sl], bias_regs[d])
            return carry
        lax.fori_loop(0, _BPW // 4, add_body, 0)

    g_copy(0, 0).start()
    g_copy(1, 1).start()
    for t in (0, 1):
        g_copy(t, t).wait()
        add_bias(t, t)
        o_copy(t, t).start()
        g_copy(t + 2, t + 2).start()

    def main_body(t4, carry):
        for j in range(_NBUF):
            t = 2 + t4 * _NBUF + j
            buf = (2 + j) % _NBUF
            nbuf = j % _NBUF
            g_copy(t, buf).wait()
            add_bias(t, buf)
            o_copy(t, buf).start()
            o_copy(t - 2, nbuf).wait()
            g_copy(t + 2, nbuf).start()
        return carry
    lax.fori_loop(0, (_T - 5) // _NBUF, main_body, 0)

    o_copy(120, 0).wait()
    g_copy(124, 0).start()
    for t in (122, 123, 124):
        j = t % _NBUF
        g_copy(t, j).wait()
        add_bias(t, j)
        o_copy(t, j).start()
    for t in range(_T - _NBUF, _T):
        o_copy(t, t % _NBUF).wait()


def kernel(zeo, syn, smis_seq, char_embed, type_embed, pe):
    idx_t = smis_seq.astype(jnp.int32).T
    pe2d = pe.reshape(_T, _D)
    out_t, out_zeo, out_syn = _embed_sc(
        zeo, syn, idx_t, char_embed, type_embed, pe2d)
    return out_t.transpose(1, 0, 2), out_zeo, out_syn
